# Initial kernel scaffold; baseline (speedup 1.0000x reference)
import jax
import jax.numpy as jnp
from jax.experimental import pallas as pl


def kernel(x, edge_index, W1, b1, W2, b2, W3, b3):
    raise NotImplementedError("write your pallas kernel here")


# speedup vs baseline: 10.9586x; 10.9586x over previous
import functools

import jax
import jax.numpy as jnp
from jax import lax
from jax.experimental import pallas as pl
from jax.experimental.pallas import tpu as pltpu
from jax.experimental.pallas import tpu_sc as plsc

NC = 2
NS = 16
NW = NC * NS

ROWS_MAIN = 632
ROWS_LAST = 10000 - 15 * ROWS_MAIN

CH = 80


def _row_partition_copy(src_ref, dst_ref, sid, src_at_zero=False):
    @pl.when(sid < NS - 1)
    def _():
        s0 = 0 if src_at_zero else sid * ROWS_MAIN
        pltpu.sync_copy(
            src_ref.at[pl.ds(s0, ROWS_MAIN)],
            dst_ref.at[pl.ds(sid * ROWS_MAIN, ROWS_MAIN)],
        )

    @pl.when(sid == NS - 1)
    def _():
        s0 = 0 if src_at_zero else (NS - 1) * ROWS_MAIN
        pltpu.sync_copy(
            src_ref.at[pl.ds(s0, ROWS_LAST)],
            dst_ref.at[pl.ds((NS - 1) * ROWS_MAIN, ROWS_LAST)],
        )


def _make_sc_spmm(n, e, d):
    ew = e // NW
    nch = ew // CH

    mesh = plsc.VectorSubcoreMesh(core_axis_name="c", subcore_axis_name="s")

    @functools.partial(
        pl.kernel,
        mesh=mesh,
        out_type=jax.ShapeDtypeStruct((NC, n, d), jnp.float32),
        scratch_types=[
            pltpu.VMEM((CH,), jnp.int32),
            pltpu.VMEM((CH,), jnp.int32),
            pltpu.VMEM((CH, d), jnp.float32),
            pltpu.VMEM_SHARED((n, d), jnp.float32),
            pltpu.SemaphoreType.DMA,
        ],
    )
    def spmm(src_hbm, dst_hbm, g_hbm, zeros_hbm, out_hbm,
             idx_s, idx_d, rows, acc, sem):
        cid = lax.axis_index("c")
        sid = lax.axis_index("s")

        _row_partition_copy(zeros_hbm, acc, sid, src_at_zero=True)
        plsc.subcore_barrier()

        base = (cid * NS + sid) * ew

        def step(j, carry):
            off = base + j * CH
            pltpu.sync_copy(src_hbm.at[pl.ds(off, CH)], idx_s)
            pltpu.sync_copy(dst_hbm.at[pl.ds(off, CH)], idx_d)
            pltpu.async_copy(g_hbm.at[idx_s], rows, sem).wait()
            pltpu.sync_copy(rows, acc.at[idx_d], add=True)
            return carry

        lax.fori_loop(0, nch, step, 0)
        plsc.subcore_barrier()
        _row_partition_copy(acc, out_hbm.at[cid], sid)

    return spmm


def _make_sc_degree(n, e):
    ew = e // NW
    nch = ew // CH

    mesh = plsc.VectorSubcoreMesh(core_axis_name="c", subcore_axis_name="s")

    @functools.partial(
        pl.kernel,
        mesh=mesh,
        out_type=jax.ShapeDtypeStruct((NC, n, 128), jnp.float32),
        scratch_types=[
            pltpu.VMEM((CH,), jnp.int32),
            pltpu.VMEM((CH, 128), jnp.float32),
            pltpu.VMEM_SHARED((n, 128), jnp.float32),
            pltpu.SemaphoreType.DMA,
        ],
    )
    def degree(dst_hbm, ones_hbm, zeros_hbm, out_hbm, idx_d, ones_v, acc, sem):
        cid = lax.axis_index("c")
        sid = lax.axis_index("s")

        _row_partition_copy(zeros_hbm, acc, sid, src_at_zero=True)
        pltpu.sync_copy(ones_hbm, ones_v)
        plsc.subcore_barrier()

        base = (cid * NS + sid) * ew

        def step(j, carry):
            off = base + j * CH
            pltpu.sync_copy(dst_hbm.at[pl.ds(off, CH)], idx_d)
            pltpu.sync_copy(ones_v, acc.at[idx_d], add=True)
            return carry

        lax.fori_loop(0, nch, step, 0)
        plsc.subcore_barrier()
        _row_partition_copy(acc, out_hbm.at[cid], sid)

    return degree


BLK = 1000


def _tc_first_body(dp_ref, x_ref, w_ref, g_ref, dinv_ref):
    deg = dp_ref[0, :, :1] + dp_ref[1, :, :1] + 1.0
    dinv = lax.rsqrt(deg)
    dinv_ref[...] = dinv
    g_ref[...] = dinv * jnp.dot(
        x_ref[...], w_ref[...], preferred_element_type=jnp.float32)


def _tc_mid_body(p_ref, g_ref, dinv_ref, b_ref, w_ref, out_ref):
    dinv = dinv_ref[...]
    h = p_ref[0] + p_ref[1] + g_ref[...]
    h = jnp.maximum(dinv * h + b_ref[...], 0.0)
    out_ref[...] = dinv * jnp.dot(
        h, w_ref[...], preferred_element_type=jnp.float32)


def _tc_last_body(d_out, p_ref, g_ref, dinv_ref, b_ref, out_ref):
    z = p_ref[0, :, :d_out] + p_ref[1, :, :d_out] + g_ref[:, :d_out]
    z = jnp.maximum(dinv_ref[...] * z + b_ref[...], 0.0)
    m = jnp.max(z, axis=1, keepdims=True)
    lse = m + jnp.log(jnp.sum(jnp.exp(z - m), axis=1, keepdims=True))
    out_ref[...] = z - lse


def _tc_first(dp, x, w):
    n, d_in = x.shape
    d_out = w.shape[1]
    grid = n // BLK
    return pl.pallas_call(
        _tc_first_body,
        grid=(grid,),
        in_specs=[
            pl.BlockSpec((NC, BLK, dp.shape[2]), lambda i: (0, i, 0)),
            pl.BlockSpec((BLK, d_in), lambda i: (i, 0)),
            pl.BlockSpec((d_in, d_out), lambda i: (0, 0)),
        ],
        out_specs=[
            pl.BlockSpec((BLK, d_out), lambda i: (i, 0)),
            pl.BlockSpec((BLK, 1), lambda i: (i, 0)),
        ],
        out_shape=[
            jax.ShapeDtypeStruct((n, d_out), jnp.float32),
            jax.ShapeDtypeStruct((n, 1), jnp.float32),
        ],
    )(dp, x, w)


def _tc_mid(p, g, dinv, b, w):
    n, d = g.shape
    d_out = w.shape[1]
    grid = n // BLK
    return pl.pallas_call(
        _tc_mid_body,
        grid=(grid,),
        in_specs=[
            pl.BlockSpec((NC, BLK, d), lambda i: (0, i, 0)),
            pl.BlockSpec((BLK, d), lambda i: (i, 0)),
            pl.BlockSpec((BLK, 1), lambda i: (i, 0)),
            pl.BlockSpec((1, d), lambda i: (0, 0)),
            pl.BlockSpec((d, d_out), lambda i: (0, 0)),
        ],
        out_specs=pl.BlockSpec((BLK, d_out), lambda i: (i, 0)),
        out_shape=jax.ShapeDtypeStruct((n, d_out), jnp.float32),
    )(p, g, dinv, b, w)


def _tc_last(p, g, dinv, b, d_out):
    n, d = g.shape
    grid = n // BLK
    return pl.pallas_call(
        functools.partial(_tc_last_body, d_out),
        grid=(grid,),
        in_specs=[
            pl.BlockSpec((NC, BLK, d), lambda i: (0, i, 0)),
            pl.BlockSpec((BLK, d), lambda i: (i, 0)),
            pl.BlockSpec((BLK, 1), lambda i: (i, 0)),
            pl.BlockSpec((1, d_out), lambda i: (0, 0)),
        ],
        out_specs=pl.BlockSpec((BLK, d_out), lambda i: (i, 0)),
        out_shape=jax.ShapeDtypeStruct((n, d_out), jnp.float32),
    )(p, g, dinv, b)


def kernel(x, edge_index, W1, b1, W2, b2, W3, b3):
    n, d_in = x.shape
    e = edge_index.shape[1]
    d_hid = W2.shape[0]
    d_out = W3.shape[1]

    src = edge_index[0]
    dst = edge_index[1]

    zeros_wide = jnp.zeros((ROWS_MAIN, max(d_in, d_hid)), jnp.float32)
    ones128 = jnp.ones((CH, 128), jnp.float32)

    sc_degree = _make_sc_degree(n, e)
    sc_spmm_h = _make_sc_spmm(n, e, d_hid)

    W3p = jnp.pad(W3, ((0, 0), (0, d_hid - d_out)))

    dp = sc_degree(dst, ones128, zeros_wide)

    g1, dinv = _tc_first(dp, x, W1)
    p1 = sc_spmm_h(src, dst, g1, zeros_wide[:, :d_hid])
    g2 = _tc_mid(p1, g1, dinv, b1.reshape(1, -1), W2)
    p2 = sc_spmm_h(src, dst, g2, zeros_wide[:, :d_hid])
    g3 = _tc_mid(p2, g2, dinv, b2.reshape(1, -1), W3p)
    p3 = sc_spmm_h(src, dst, g3, zeros_wide[:, :d_hid])
    return _tc_last(p3, g3, dinv, b3.reshape(1, -1), d_out)


# speedup vs baseline: 20.3738x; 1.8592x over previous
import functools

import jax
import jax.numpy as jnp
from jax import lax
from jax.experimental import pallas as pl
from jax.experimental.pallas import tpu as pltpu
from jax.experimental.pallas import tpu_sc as plsc

NC = 2
NS = 16
NW = NC * NS

ROWS_MAIN = 632
ROWS_LAST = 10000 - 15 * ROWS_MAIN

CH = 128


def _row_partition_copy(src_ref, dst_ref, sid, src_at_zero=False):
    @pl.when(sid < NS - 1)
    def _():
        s0 = 0 if src_at_zero else sid * ROWS_MAIN
        pltpu.sync_copy(
            src_ref.at[pl.ds(s0, ROWS_MAIN)],
            dst_ref.at[pl.ds(sid * ROWS_MAIN, ROWS_MAIN)],
        )

    @pl.when(sid == NS - 1)
    def _():
        s0 = 0 if src_at_zero else (NS - 1) * ROWS_MAIN
        pltpu.sync_copy(
            src_ref.at[pl.ds(s0, ROWS_LAST)],
            dst_ref.at[pl.ds((NS - 1) * ROWS_MAIN, ROWS_LAST)],
        )


def _chunk_counts(e):
    nch_total = e // CH
    per_w = nch_total // NW
    extra = nch_total - per_w * NW
    return nch_total, per_w, extra


def _make_sc_spmm(n, e, d):
    _, per_w, extra = _chunk_counts(e)

    mesh = plsc.VectorSubcoreMesh(core_axis_name="c", subcore_axis_name="s")

    @functools.partial(
        pl.kernel,
        mesh=mesh,
        out_type=jax.ShapeDtypeStruct((NC, n, d), jnp.float32),
        scratch_types=[
            pltpu.VMEM((CH,), jnp.int32),
            pltpu.VMEM((CH,), jnp.int32),
            pltpu.VMEM((CH,), jnp.int32),
            pltpu.VMEM((CH,), jnp.int32),
            pltpu.VMEM((CH, d), jnp.float32),
            pltpu.VMEM((CH, d), jnp.float32),
            pltpu.VMEM_SHARED((n, d), jnp.float32),
            pltpu.SemaphoreType.DMA,
            pltpu.SemaphoreType.DMA,
        ],
    )
    def spmm(src_hbm, dst_hbm, g_hbm, zeros_hbm, out_hbm,
             is0, id0, is1, id1, rows0, rows1, acc, sem0, sem1):
        cid = lax.axis_index("c")
        sid = lax.axis_index("s")
        wid = cid * NS + sid

        _row_partition_copy(zeros_hbm, acc, sid, src_at_zero=True)
        plsc.subcore_barrier()

        def load_idx(chunk, is_ref, id_ref):
            off = chunk * CH
            pltpu.sync_copy(src_hbm.at[pl.ds(off, CH)], is_ref)
            pltpu.sync_copy(dst_hbm.at[pl.ds(off, CH)], id_ref)

        load_idx(wid, is0, id0)
        pltpu.async_copy(g_hbm.at[is0], rows0, sem0)
        load_idx(wid + NW, is1, id1)
        pltpu.async_copy(g_hbm.at[is1], rows1, sem1)

        def step(k, carry):
            pltpu.make_async_copy(g_hbm.at[is0], rows0, sem0).wait()
            pltpu.sync_copy(rows0, acc.at[id0], add=True)
            load_idx(wid + (2 * k + 2) * NW, is0, id0)
            pltpu.async_copy(g_hbm.at[is0], rows0, sem0)

            pltpu.make_async_copy(g_hbm.at[is1], rows1, sem1).wait()
            pltpu.sync_copy(rows1, acc.at[id1], add=True)
            load_idx(wid + (2 * k + 3) * NW, is1, id1)
            pltpu.async_copy(g_hbm.at[is1], rows1, sem1)
            return carry

        lax.fori_loop(0, per_w // 2 - 1, step, 0)

        pltpu.make_async_copy(g_hbm.at[is0], rows0, sem0).wait()
        pltpu.sync_copy(rows0, acc.at[id0], add=True)
        pltpu.make_async_copy(g_hbm.at[is1], rows1, sem1).wait()
        pltpu.sync_copy(rows1, acc.at[id1], add=True)

        @pl.when(wid < extra)
        def _():
            load_idx(per_w * NW + wid, is0, id0)
            pltpu.async_copy(g_hbm.at[is0], rows0, sem0).wait()
            pltpu.sync_copy(rows0, acc.at[id0], add=True)

        plsc.subcore_barrier()
        _row_partition_copy(acc, out_hbm.at[cid], sid)

    return spmm


def _make_sc_degree(n, e):
    _, per_w, extra = _chunk_counts(e)

    mesh = plsc.VectorSubcoreMesh(core_axis_name="c", subcore_axis_name="s")

    @functools.partial(
        pl.kernel,
        mesh=mesh,
        out_type=jax.ShapeDtypeStruct((NC, n, 128), jnp.float32),
        scratch_types=[
            pltpu.VMEM((CH,), jnp.int32),
            pltpu.VMEM((CH,), jnp.int32),
            pltpu.VMEM((CH, 128), jnp.float32),
            pltpu.VMEM_SHARED((n, 128), jnp.float32),
            pltpu.SemaphoreType.DMA,
            pltpu.SemaphoreType.DMA,
        ],
    )
    def degree(dst_hbm, ones_hbm, zeros_hbm, out_hbm,
               id0, id1, ones_v, acc, sem0, sem1):
        cid = lax.axis_index("c")
        sid = lax.axis_index("s")
        wid = cid * NS + sid

        _row_partition_copy(zeros_hbm, acc, sid, src_at_zero=True)
        pltpu.sync_copy(ones_hbm, ones_v)
        plsc.subcore_barrier()

        pltpu.sync_copy(dst_hbm.at[pl.ds(wid * CH, CH)], id0)
        pltpu.async_copy(ones_v, acc.at[id0], sem0, add=True)
        pltpu.sync_copy(dst_hbm.at[pl.ds((wid + NW) * CH, CH)], id1)
        pltpu.async_copy(ones_v, acc.at[id1], sem1, add=True)

        def step(k, carry):
            pltpu.make_async_copy(ones_v, acc.at[id0], sem0).wait()
            pltpu.sync_copy(dst_hbm.at[pl.ds((wid + (2 * k + 2) * NW) * CH, CH)], id0)
            pltpu.async_copy(ones_v, acc.at[id0], sem0, add=True)

            pltpu.make_async_copy(ones_v, acc.at[id1], sem1).wait()
            pltpu.sync_copy(dst_hbm.at[pl.ds((wid + (2 * k + 3) * NW) * CH, CH)], id1)
            pltpu.async_copy(ones_v, acc.at[id1], sem1, add=True)
            return carry

        lax.fori_loop(0, per_w // 2 - 1, step, 0)
        pltpu.make_async_copy(ones_v, acc.at[id0], sem0).wait()
        pltpu.make_async_copy(ones_v, acc.at[id1], sem1).wait()

        @pl.when(wid < extra)
        def _():
            pltpu.sync_copy(dst_hbm.at[pl.ds((per_w * NW + wid) * CH, CH)], id0)
            pltpu.sync_copy(ones_v, acc.at[id0], add=True)

        plsc.subcore_barrier()
        _row_partition_copy(acc, out_hbm.at[cid], sid)

    return degree


BLK = 1000


def _tc_mm_body(x_ref, w_ref, out_ref):
    out_ref[...] = jnp.dot(
        x_ref[...], w_ref[...], preferred_element_type=jnp.float32)


def _tc_scale_body(dp_ref, h_ref, g_ref, dinv_ref):
    deg = dp_ref[0, :, :1] + dp_ref[1, :, :1] + 1.0
    dinv = lax.rsqrt(deg)
    dinv_ref[...] = dinv
    g_ref[...] = dinv * h_ref[...]


def _tc_mid_body(p_ref, g_ref, dinv_ref, b_ref, w_ref, out_ref):
    dinv = dinv_ref[...]
    h = p_ref[0] + p_ref[1] + g_ref[...]
    h = jnp.maximum(dinv * h + b_ref[...], 0.0)
    out_ref[...] = dinv * jnp.dot(
        h, w_ref[...], preferred_element_type=jnp.float32)


def _tc_last_body(d_out, p_ref, g_ref, dinv_ref, b_ref, out_ref):
    z = p_ref[0, :, :d_out] + p_ref[1, :, :d_out] + g_ref[:, :d_out]
    z = jnp.maximum(dinv_ref[...] * z + b_ref[...], 0.0)
    m = jnp.max(z, axis=1, keepdims=True)
    lse = m + jnp.log(jnp.sum(jnp.exp(z - m), axis=1, keepdims=True))
    out_ref[...] = z - lse


def _tc_mm(x, w):
    n, d_in = x.shape
    d_out = w.shape[1]
    return pl.pallas_call(
        _tc_mm_body,
        grid=(n // BLK,),
        in_specs=[
            pl.BlockSpec((BLK, d_in), lambda i: (i, 0)),
            pl.BlockSpec((d_in, d_out), lambda i: (0, 0)),
        ],
        out_specs=pl.BlockSpec((BLK, d_out), lambda i: (i, 0)),
        out_shape=jax.ShapeDtypeStruct((n, d_out), jnp.float32),
    )(x, w)


def _tc_scale(dp, h):
    n, d = h.shape
    return pl.pallas_call(
        _tc_scale_body,
        grid=(n // BLK,),
        in_specs=[
            pl.BlockSpec((NC, BLK, dp.shape[2]), lambda i: (0, i, 0)),
            pl.BlockSpec((BLK, d), lambda i: (i, 0)),
        ],
        out_specs=[
            pl.BlockSpec((BLK, d), lambda i: (i, 0)),
            pl.BlockSpec((BLK, 1), lambda i: (i, 0)),
        ],
        out_shape=[
            jax.ShapeDtypeStruct((n, d), jnp.float32),
            jax.ShapeDtypeStruct((n, 1), jnp.float32),
        ],
    )(dp, h)


def _tc_mid(p, g, dinv, b, w):
    n, d = g.shape
    d_out = w.shape[1]
    return pl.pallas_call(
        _tc_mid_body,
        grid=(n // BLK,),
        in_specs=[
            pl.BlockSpec((NC, BLK, d), lambda i: (0, i, 0)),
            pl.BlockSpec((BLK, d), lambda i: (i, 0)),
            pl.BlockSpec((BLK, 1), lambda i: (i, 0)),
            pl.BlockSpec((1, d), lambda i: (0, 0)),
            pl.BlockSpec((d, d_out), lambda i: (0, 0)),
        ],
        out_specs=pl.BlockSpec((BLK, d_out), lambda i: (i, 0)),
        out_shape=jax.ShapeDtypeStruct((n, d_out), jnp.float32),
    )(p, g, dinv, b, w)


def _tc_last(p, g, dinv, b, d_out):
    n, d = g.shape
    return pl.pallas_call(
        functools.partial(_tc_last_body, d_out),
        grid=(n // BLK,),
        in_specs=[
            pl.BlockSpec((NC, BLK, d), lambda i: (0, i, 0)),
            pl.BlockSpec((BLK, d), lambda i: (i, 0)),
            pl.BlockSpec((BLK, 1), lambda i: (i, 0)),
            pl.BlockSpec((1, d_out), lambda i: (0, 0)),
        ],
        out_specs=pl.BlockSpec((BLK, d_out), lambda i: (i, 0)),
        out_shape=jax.ShapeDtypeStruct((n, d_out), jnp.float32),
    )(p, g, dinv, b)


def kernel(x, edge_index, W1, b1, W2, b2, W3, b3):
    n, d_in = x.shape
    e = edge_index.shape[1]
    d_hid = W2.shape[0]
    d_out = W3.shape[1]

    src = edge_index[0]
    dst = edge_index[1]

    zeros_wide = jnp.zeros((ROWS_MAIN, max(d_in, d_hid)), jnp.float32)
    ones128 = jnp.ones((CH, 128), jnp.float32)

    sc_degree = _make_sc_degree(n, e)
    sc_spmm_h = _make_sc_spmm(n, e, d_hid)

    W3p = jnp.pad(W3, ((0, 0), (0, d_hid - d_out)))

    dp = sc_degree(dst, ones128, zeros_wide)
    h1 = _tc_mm(x, W1)
    g1, dinv = _tc_scale(dp, h1)
    p1 = sc_spmm_h(src, dst, g1, zeros_wide)
    g2 = _tc_mid(p1, g1, dinv, b1.reshape(1, -1), W2)
    p2 = sc_spmm_h(src, dst, g2, zeros_wide)
    g3 = _tc_mid(p2, g2, dinv, b2.reshape(1, -1), W3p)
    p3 = sc_spmm_h(src, dst, g3, zeros_wide)
    return _tc_last(p3, g3, dinv, b3.reshape(1, -1), d_out)
